# baseline (speedup 1.0000x reference)
import jax
import jax.numpy as jnp
from jax.experimental import pallas as pl
from jax.experimental.pallas import tpu as pltpu

_B = 1024
_L = 64
_LA = _L + 1
_KBIG = 10000
_NSPLIT = 5
_KSUB = 400
_NSTEP = _KBIG // (_NSPLIT * _KSUB)


def _body(*refs):
    (xd_ref, xc_ref, xco_ref,
     xm0, xm1, xm2, xm3, xm4, xp0, xp1, xp2, xp3, xp4,
     wd_ref, wc_ref, wco_ref,
     wm0, wm1, wm2, wm3, wm4, wp0, wp1, wp2, wp3, wp4,
     out_ref, ym_acc, yp_acc) = refs
    step = pl.program_id(0)
    dn = (((1,), (0,)), ((), ()))

    def part(w_ref, x_ref):
        xb = x_ref[...].astype(jnp.bfloat16)
        return jax.lax.dot_general(w_ref[...], xb, dn,
                                   preferred_element_type=jnp.float32)

    def acc_field(w_refs, x_refs):
        y = None
        for w_ref, x_ref in zip(w_refs, x_refs):
            p = part(w_ref.at[0], x_ref)
            y = p if y is None else y + p
        return y

    pm = acc_field((wm0, wm1, wm2, wm3, wm4), (xm0, xm1, xm2, xm3, xm4))
    pp = acc_field((wp0, wp1, wp2, wp3, wp4), (xp0, xp1, xp2, xp3, xp4))

    @pl.when(step == 0)
    def _():
        ym_acc[...] = pm
        yp_acc[...] = pp

    @pl.when(step != 0)
    def _():
        ym_acc[...] += pm
        yp_acc[...] += pp

    @pl.when(step == _NSTEP - 1)
    def _():
        yd = part(wd_ref, xd_ref)
        yc = part(wc_ref, xc_ref)
        yco = part(wco_ref, xco_ref)
        ym = ym_acc[...]
        yp = yp_acc[...]

        def mean_div(y, s):
            nz = s != 0.0
            return jnp.where(nz, y / jnp.where(nz, s, 1.0), y)

        sd, sm, sc, sp, sco = (y[_L:_L + 1, :]
                               for y in (yd, ym, yc, yp, yco))
        yd, ym, yc, yp, yco = (y[:_L, :] for y in (yd, ym, yc, yp, yco))

        yd = mean_div(yd, sd)
        yd = mean_div(yd, sm)
        yc = mean_div(yc, sc)
        yp = mean_div(yp, sp)
        yco = mean_div(yco, sco)

        out_ref[...] = jnp.concatenate((yd, ym, yc, yp, yco), axis=0)


def _aug(W):
    wa = jnp.concatenate([W, jnp.ones((1, W.shape[1]), jnp.float32)], axis=0)
    return wa.astype(jnp.bfloat16)


def kernel(decade_idxs, movie_idxs, category_idxs, person_idxs, company_idxs,
           W_decade, W_movie, W_category, W_person, W_company):
    xd, xm, xc, xp, xco = (x.T for x in (
        decade_idxs, movie_idxs, category_idxs, person_idxs, company_idxs))
    wd, wm, wc, wp, wco = (_aug(W) for W in (
        W_decade, W_movie, W_category, W_person, W_company))
    nchunk = _NSTEP * _NSPLIT
    wm = wm.reshape(_LA, nchunk, _KSUB).transpose(1, 0, 2)
    wp = wp.reshape(_LA, nchunk, _KSUB).transpose(1, 0, 2)

    grid = (_NSTEP,)

    def xsub_spec(j):
        return pl.BlockSpec((_KSUB, _B),
                            lambda i, j=j: (i * _NSPLIT + j, 0))

    def wsub_spec(j):
        return pl.BlockSpec((1, _LA, _KSUB),
                            lambda i, j=j: (i * _NSPLIT + j, 0, 0))

    in_specs = (
        [pl.BlockSpec((16, _B), lambda i: (0, 0)),
         pl.BlockSpec((1000, _B), lambda i: (0, 0)),
         pl.BlockSpec((1000, _B), lambda i: (0, 0))]
        + [xsub_spec(j) for j in range(_NSPLIT)] * 2
        + [pl.BlockSpec((_LA, 16), lambda i: (0, 0)),
           pl.BlockSpec((_LA, 1000), lambda i: (0, 0)),
           pl.BlockSpec((_LA, 1000), lambda i: (0, 0))]
        + [wsub_spec(j) for j in range(_NSPLIT)] * 2
    )
    out_t = pl.pallas_call(
        _body,
        grid=grid,
        in_specs=in_specs,
        out_specs=pl.BlockSpec((5 * _L, _B), lambda i: (0, 0)),
        out_shape=jax.ShapeDtypeStruct((5 * _L, _B), jnp.float32),
        scratch_shapes=[pltpu.VMEM((_LA, _B), jnp.float32)] * 2,
    )(xd, xc, xco, xm, xm, xm, xm, xm, xp, xp, xp, xp, xp,
      wd, wc, wco, wm, wm, wm, wm, wm, wp, wp, wp, wp, wp)
    return out_t.T


# speedup vs baseline: 1.0631x; 1.0631x over previous
import jax
import jax.numpy as jnp
from jax.experimental import pallas as pl
from jax.experimental.pallas import tpu as pltpu

_B = 1024
_L = 64
_LA = _L + 1
_KBIG = 10000
_KC = 1000
_NCH = _KBIG // _KC
_RING = 4


def _body(xd_hbm, xm_hbm, xc_hbm, xp_hbm, xco_hbm,
          wd_ref, wm_ref, wc_ref, wp_ref, wco_ref,
          out_ref,
          bd, bm, bc, bp, bco, sems, ssem):
    dn = (((1,), (0,)), ((), ()))

    def start_big(hbm, buf, col, k):
        pltpu.make_async_copy(
            hbm.at[pl.ds(k * _KC, _KC), :],
            buf.at[k % _RING],
            sems.at[k % _RING, col],
        ).start()

    def wait_big(hbm, buf, col, k):
        pltpu.make_async_copy(
            hbm.at[pl.ds(k * _KC, _KC), :],
            buf.at[k % _RING],
            sems.at[k % _RING, col],
        ).wait()

    pltpu.make_async_copy(xc_hbm, bc, ssem.at[0]).start()
    pltpu.make_async_copy(xco_hbm, bco, ssem.at[1]).start()
    pltpu.make_async_copy(xd_hbm, bd, ssem.at[2]).start()
    for k in range(_RING):
        start_big(xm_hbm, bm, 0, k)
        start_big(xp_hbm, bp, 1, k)

    def part(w, x_ref):
        xb = x_ref[...].astype(jnp.bfloat16)
        return jax.lax.dot_general(w, xb, dn,
                                   preferred_element_type=jnp.float32)

    ym = yp = yd = yc = yco = None
    for k in range(_NCH):
        wait_big(xm_hbm, bm, 0, k)
        pm = part(wm_ref[k], bm.at[k % _RING])
        ym = pm if ym is None else ym + pm
        if k + _RING < _NCH:
            start_big(xm_hbm, bm, 0, k + _RING)

        wait_big(xp_hbm, bp, 1, k)
        pp = part(wp_ref[k], bp.at[k % _RING])
        yp = pp if yp is None else yp + pp
        if k + _RING < _NCH:
            start_big(xp_hbm, bp, 1, k + _RING)

        if k == 2:
            pltpu.make_async_copy(xc_hbm, bc, ssem.at[0]).wait()
            yc = part(wc_ref[...], bc)
            pltpu.make_async_copy(xco_hbm, bco, ssem.at[1]).wait()
            yco = part(wco_ref[...], bco)
            pltpu.make_async_copy(xd_hbm, bd, ssem.at[2]).wait()
            yd = part(wd_ref[...], bd)

    def mean_div(y, s):
        nz = s != 0.0
        return jnp.where(nz, y / jnp.where(nz, s, 1.0), y)

    sd, sm, sc, sp, sco = (y[_L:_L + 1, :] for y in (yd, ym, yc, yp, yco))
    yd, ym, yc, yp, yco = (y[:_L, :] for y in (yd, ym, yc, yp, yco))

    yd = mean_div(yd, sd)
    yd = mean_div(yd, sm)
    yc = mean_div(yc, sc)
    yp = mean_div(yp, sp)
    yco = mean_div(yco, sco)

    out_ref[...] = jnp.concatenate((yd, ym, yc, yp, yco), axis=0)


def _aug(W):
    wa = jnp.concatenate([W, jnp.ones((1, W.shape[1]), jnp.float32)], axis=0)
    return wa.astype(jnp.bfloat16)


def kernel(decade_idxs, movie_idxs, category_idxs, person_idxs, company_idxs,
           W_decade, W_movie, W_category, W_person, W_company):
    xd, xm, xc, xp, xco = (x.T for x in (
        decade_idxs, movie_idxs, category_idxs, person_idxs, company_idxs))
    wd, wm, wc, wp, wco = (_aug(W) for W in (
        W_decade, W_movie, W_category, W_person, W_company))
    wm = wm.reshape(_LA, _NCH, _KC).transpose(1, 0, 2)
    wp = wp.reshape(_LA, _NCH, _KC).transpose(1, 0, 2)

    any_spec = pl.BlockSpec(memory_space=pl.ANY)
    in_specs = [
        any_spec, any_spec, any_spec, any_spec, any_spec,
        pl.BlockSpec((_LA, 16), lambda: (0, 0)),
        pl.BlockSpec((_NCH, _LA, _KC), lambda: (0, 0, 0)),
        pl.BlockSpec((_LA, 1000), lambda: (0, 0)),
        pl.BlockSpec((_NCH, _LA, _KC), lambda: (0, 0, 0)),
        pl.BlockSpec((_LA, 1000), lambda: (0, 0)),
    ]
    scratch_shapes = [
        pltpu.VMEM((16, _B), jnp.int32),
        pltpu.VMEM((_RING, _KC, _B), jnp.int32),
        pltpu.VMEM((1000, _B), jnp.int32),
        pltpu.VMEM((_RING, _KC, _B), jnp.int32),
        pltpu.VMEM((1000, _B), jnp.int32),
        pltpu.SemaphoreType.DMA((_RING, 2)),
        pltpu.SemaphoreType.DMA((3,)),
    ]
    out_t = pl.pallas_call(
        _body,
        in_specs=in_specs,
        out_specs=pl.BlockSpec((5 * _L, _B), lambda: (0, 0)),
        out_shape=jax.ShapeDtypeStruct((5 * _L, _B), jnp.float32),
        scratch_shapes=scratch_shapes,
    )(xd, xm, xc, xp, xco, wd, wm, wc, wp, wco)
    return out_t.T


# speedup vs baseline: 1.2470x; 1.1729x over previous
import jax
import jax.numpy as jnp
from jax.experimental import pallas as pl
from jax.experimental.pallas import tpu as pltpu

_B = 1024
_L = 64
_LA = _L + 1
_KBIG = 10000
_KC = 1024
_RING = 4
_CHUNKS = tuple((k * _KC, min(_KC, _KBIG - k * _KC))
                for k in range((_KBIG + _KC - 1) // _KC))
_NCH = len(_CHUNKS)


def _body(xd_hbm, xm_hbm, xc_hbm, xp_hbm, xco_hbm,
          wd_ref, wm_ref, wc_ref, wp_ref, wco_ref,
          out_ref,
          bd, bm, bc, bp, bco, sems, ssem):
    dn = (((1,), (0,)), ((), ()))

    def start_big(hbm, buf, col, k):
        off, width = _CHUNKS[k]
        pltpu.make_async_copy(
            hbm.at[pl.ds(off, width), :],
            buf.at[k % _RING, pl.ds(0, width), :],
            sems.at[k % _RING, col],
        ).start()

    def wait_big(hbm, buf, col, k):
        off, width = _CHUNKS[k]
        pltpu.make_async_copy(
            hbm.at[pl.ds(off, width), :],
            buf.at[k % _RING, pl.ds(0, width), :],
            sems.at[k % _RING, col],
        ).wait()

    pltpu.make_async_copy(xc_hbm, bc, ssem.at[0]).start()
    pltpu.make_async_copy(xco_hbm, bco, ssem.at[1]).start()
    pltpu.make_async_copy(xd_hbm, bd, ssem.at[2]).start()
    for k in range(_RING):
        start_big(xm_hbm, bm, 0, k)
        start_big(xp_hbm, bp, 1, k)

    def part(w, x):
        return jax.lax.dot_general(w, x.astype(jnp.bfloat16), dn,
                                   preferred_element_type=jnp.float32)

    ym = yp = yd = yc = yco = None
    for k in range(_NCH):
        off, width = _CHUNKS[k]
        wait_big(xm_hbm, bm, 0, k)
        pm = part(wm_ref[:, pl.ds(off, width)], bm[k % _RING, pl.ds(0, width), :])
        ym = pm if ym is None else ym + pm
        if k + _RING < _NCH:
            start_big(xm_hbm, bm, 0, k + _RING)

        wait_big(xp_hbm, bp, 1, k)
        pp = part(wp_ref[:, pl.ds(off, width)], bp[k % _RING, pl.ds(0, width), :])
        yp = pp if yp is None else yp + pp
        if k + _RING < _NCH:
            start_big(xp_hbm, bp, 1, k + _RING)

        if k == 2:
            pltpu.make_async_copy(xc_hbm, bc, ssem.at[0]).wait()
            yc = part(wc_ref[...], bc[...])
            pltpu.make_async_copy(xco_hbm, bco, ssem.at[1]).wait()
            yco = part(wco_ref[...], bco[...])
            pltpu.make_async_copy(xd_hbm, bd, ssem.at[2]).wait()
            yd = part(wd_ref[...], bd[...])

    def mean_div(y, s):
        nz = s != 0.0
        return jnp.where(nz, y / jnp.where(nz, s, 1.0), y)

    sd, sm, sc, sp, sco = (y[_L:_L + 1, :] for y in (yd, ym, yc, yp, yco))
    yd, ym, yc, yp, yco = (y[:_L, :] for y in (yd, ym, yc, yp, yco))

    yd = mean_div(yd, sd)
    yd = mean_div(yd, sm)
    yc = mean_div(yc, sc)
    yp = mean_div(yp, sp)
    yco = mean_div(yco, sco)

    out_ref[...] = jnp.concatenate((yd, ym, yc, yp, yco), axis=0)


def _aug(W):
    wa = jnp.concatenate([W, jnp.ones((1, W.shape[1]), jnp.float32)], axis=0)
    return wa.astype(jnp.bfloat16)


def kernel(decade_idxs, movie_idxs, category_idxs, person_idxs, company_idxs,
           W_decade, W_movie, W_category, W_person, W_company):
    xd, xm, xc, xp, xco = (x.T for x in (
        decade_idxs, movie_idxs, category_idxs, person_idxs, company_idxs))
    wd, wm, wc, wp, wco = (_aug(W) for W in (
        W_decade, W_movie, W_category, W_person, W_company))

    any_spec = pl.BlockSpec(memory_space=pl.ANY)
    in_specs = [
        any_spec, any_spec, any_spec, any_spec, any_spec,
        pl.BlockSpec((_LA, 16), lambda: (0, 0)),
        pl.BlockSpec((_LA, _KBIG), lambda: (0, 0)),
        pl.BlockSpec((_LA, 1000), lambda: (0, 0)),
        pl.BlockSpec((_LA, _KBIG), lambda: (0, 0)),
        pl.BlockSpec((_LA, 1000), lambda: (0, 0)),
    ]
    scratch_shapes = [
        pltpu.VMEM((16, _B), jnp.int32),
        pltpu.VMEM((_RING, _KC, _B), jnp.int32),
        pltpu.VMEM((1000, _B), jnp.int32),
        pltpu.VMEM((_RING, _KC, _B), jnp.int32),
        pltpu.VMEM((1000, _B), jnp.int32),
        pltpu.SemaphoreType.DMA((_RING, 2)),
        pltpu.SemaphoreType.DMA((3,)),
    ]
    out_t = pl.pallas_call(
        _body,
        in_specs=in_specs,
        out_specs=pl.BlockSpec((5 * _L, _B), lambda: (0, 0)),
        out_shape=jax.ShapeDtypeStruct((5 * _L, _B), jnp.float32),
        scratch_shapes=scratch_shapes,
    )(xd, xm, xc, xp, xco, wd, wm, wc, wp, wco)
    return out_t.T


# speedup vs baseline: 1.5519x; 1.2445x over previous
import jax
import jax.numpy as jnp
from jax.experimental import pallas as pl
from jax.experimental.pallas import tpu as pltpu

_B = 1024
_L = 64
_KBIG = 10000
_KC = 1024
_RING = 4
_CHUNKS = tuple((k * _KC, min(_KC, _KBIG - k * _KC))
                for k in range((_KBIG + _KC - 1) // _KC))
_NCH = len(_CHUNKS)


def _body(xd_hbm, xm_hbm, xc_hbm, xp_hbm, xco_hbm,
          wd_ref, wm_ref, wc_ref, wp_ref, wco_ref,
          out_ref,
          bd, bm, bc, bp, bco, sems, ssem):
    dn = (((1,), (0,)), ((), ()))

    def start_big(hbm, buf, col, k):
        off, width = _CHUNKS[k]
        pltpu.make_async_copy(
            hbm.at[pl.ds(off, width), :],
            buf.at[k % _RING, pl.ds(0, width), :],
            sems.at[k % _RING, col],
        ).start()

    def wait_big(hbm, buf, col, k):
        off, width = _CHUNKS[k]
        pltpu.make_async_copy(
            hbm.at[pl.ds(off, width), :],
            buf.at[k % _RING, pl.ds(0, width), :],
            sems.at[k % _RING, col],
        ).wait()

    pltpu.make_async_copy(xc_hbm, bc, ssem.at[0]).start()
    pltpu.make_async_copy(xco_hbm, bco, ssem.at[1]).start()
    pltpu.make_async_copy(xd_hbm, bd, ssem.at[2]).start()
    for k in range(_RING):
        start_big(xm_hbm, bm, 0, k)
        start_big(xp_hbm, bp, 1, k)

    def part(w, xi):
        y = jax.lax.dot_general(
            w.astype(jnp.bfloat16), xi.astype(jnp.bfloat16), dn,
            preferred_element_type=jnp.float32)
        s = jnp.sum(xi, axis=0, keepdims=True)
        return y, s

    ym = sm = yp = sp = None
    for k in range(_NCH):
        off, width = _CHUNKS[k]
        wait_big(xm_hbm, bm, 0, k)
        pm, qm = part(wm_ref[:, pl.ds(off, width)],
                      bm[k % _RING, pl.ds(0, width), :])
        ym = pm if ym is None else ym + pm
        sm = qm if sm is None else sm + qm
        if k + _RING < _NCH:
            start_big(xm_hbm, bm, 0, k + _RING)

        wait_big(xp_hbm, bp, 1, k)
        pp, qp = part(wp_ref[:, pl.ds(off, width)],
                      bp[k % _RING, pl.ds(0, width), :])
        yp = pp if yp is None else yp + pp
        sp = qp if sp is None else sp + qp
        if k + _RING < _NCH:
            start_big(xp_hbm, bp, 1, k + _RING)

        if k == 2:
            pltpu.make_async_copy(xc_hbm, bc, ssem.at[0]).wait()
            yc, sc = part(wc_ref[...], bc[...])
            pltpu.make_async_copy(xco_hbm, bco, ssem.at[1]).wait()
            yco, sco = part(wco_ref[...], bco[...])
            pltpu.make_async_copy(xd_hbm, bd, ssem.at[2]).wait()
            yd, sd = part(wd_ref[...], bd[...])

    def mean_div(y, s):
        nz = s != 0
        sf = jnp.where(nz, s, 1).astype(jnp.float32)
        return jnp.where(nz, y / sf, y)

    yd = mean_div(yd, sd)
    yd = mean_div(yd, sm)
    yc = mean_div(yc, sc)
    yp = mean_div(yp, sp)
    yco = mean_div(yco, sco)

    out_ref[...] = jnp.concatenate((yd, ym, yc, yp, yco), axis=0)


def kernel(decade_idxs, movie_idxs, category_idxs, person_idxs, company_idxs,
           W_decade, W_movie, W_category, W_person, W_company):
    xd, xm, xc, xp, xco = (x.T for x in (
        decade_idxs, movie_idxs, category_idxs, person_idxs, company_idxs))

    any_spec = pl.BlockSpec(memory_space=pl.ANY)
    in_specs = [
        any_spec, any_spec, any_spec, any_spec, any_spec,
        pl.BlockSpec((_L, 16), lambda: (0, 0)),
        pl.BlockSpec((_L, _KBIG), lambda: (0, 0)),
        pl.BlockSpec((_L, 1000), lambda: (0, 0)),
        pl.BlockSpec((_L, _KBIG), lambda: (0, 0)),
        pl.BlockSpec((_L, 1000), lambda: (0, 0)),
    ]
    scratch_shapes = [
        pltpu.VMEM((16, _B), jnp.int32),
        pltpu.VMEM((_RING, _KC, _B), jnp.int32),
        pltpu.VMEM((1000, _B), jnp.int32),
        pltpu.VMEM((_RING, _KC, _B), jnp.int32),
        pltpu.VMEM((1000, _B), jnp.int32),
        pltpu.SemaphoreType.DMA((_RING, 2)),
        pltpu.SemaphoreType.DMA((3,)),
    ]
    out_t = pl.pallas_call(
        _body,
        in_specs=in_specs,
        out_specs=pl.BlockSpec((5 * _L, _B), lambda: (0, 0)),
        out_shape=jax.ShapeDtypeStruct((5 * _L, _B), jnp.float32),
        scratch_shapes=scratch_shapes,
    )(xd, xm, xc, xp, xco,
      W_decade, W_movie, W_category, W_person, W_company)
    return out_t.T
